# baseline (speedup 1.0000x reference)
import jax
import jax.numpy as jnp
import numpy as np
from jax import lax
from jax.experimental import pallas as pl
from jax.experimental.pallas import tpu as pltpu
from jax.experimental.pallas import tpu_sc as plsc

BS = 32
N = 8192
D = 256
AD = 11
K = 900
KP = 904
L = 16
NV = N // L
KV = KP // L + 1
CH = 128
NCH = K // CH
REM = KP - NCH * CH
MSB = np.int32(-2147483648)


def _desc_key(bits):
    asc = jnp.where(bits < 0, ~bits, bits | MSB)
    return ~asc


def _key_to_f32(key):
    asc = ~key
    bits = jnp.where(asc < 0, asc & np.int32(0x7FFFFFFF), ~asc)
    return plsc.bitcast(bits, jnp.float32)


def _digit(key, shift):
    return lax.shift_right_logical(key, np.int32(shift)) & np.int32(255)


def _sc_body(conf_hbm, feat_hbm, anch_hbm, koff_hbm,
             conf_out, feat_out, anch_out,
             row_v, key0, key1, idx0, idx1, hist, base, vals, gidx,
             koff_v, fbuf, abuf, fsem, asem):
    b = lax.axis_index("s") * 2 + lax.axis_index("c")

    pltpu.sync_copy(conf_hbm.at[b], row_v)
    pltpu.sync_copy(koff_hbm, koff_v)

    zeros = jnp.zeros((L,), jnp.int32)
    for j in range(256 // L):
        hist[pl.ds(j * L, L)] = zeros

    def build(r, _):
        for j in range(8):
            bits = plsc.bitcast(row_v[r, pl.ds(j * L, L)], jnp.int32)
            key = _desc_key(bits)
            sl = pl.ds(r * 128 + j * L, L)
            key0[sl] = key
            idx0[sl] = (r * 128 + j * L) + lax.iota(jnp.int32, L)
            d = _digit(key, 0)
            cnt, last = plsc.scan_count(d)
            plsc.addupdate_scatter(hist, [d], cnt, mask=last)
        return 0

    lax.fori_loop(0, N // 128, build, 0)

    def prefix_and_zero():
        carry = jnp.int32(0)
        for j in range(256 // L):
            sl = pl.ds(j * L, L)
            c = hist[sl]
            incl = plsc.cumsum(c)
            base[sl] = incl - c + carry
            hist[sl] = zeros
            carry = carry + jnp.sum(c)

    def radix_pass(src_k, src_i, dst_k, dst_i, shift, next_shift):
        prefix_and_zero()

        def step(i, _):
            sl = pl.ds(i * L, L)
            key = src_k[sl]
            val = src_i[sl]
            d = _digit(key, shift)
            cnt, last = plsc.scan_count(d)
            pos = plsc.load_gather(base, [d]) + cnt - 1
            plsc.store_scatter(dst_k, [pos], key)
            plsc.store_scatter(dst_i, [pos], val)
            plsc.addupdate_scatter(base, [d], cnt, mask=last)
            if next_shift is not None:
                d2 = _digit(key, next_shift)
                cnt2, last2 = plsc.scan_count(d2)
                plsc.addupdate_scatter(hist, [d2], cnt2, mask=last2)
            return 0

        lax.fori_loop(0, NV, step, 0)

    radix_pass(key0, idx0, key1, idx1, 0, 8)
    radix_pass(key1, idx1, key0, idx0, 8, 16)
    radix_pass(key0, idx0, key1, idx1, 16, 24)
    radix_pass(key1, idx1, key0, idx0, 24, None)

    off = koff_v[pl.ds(0, L)] + b * N
    for i in range(KV):
        sl = pl.ds(i * L, L)
        r, c = i // 8, (i % 8) * L
        vals[r, pl.ds(c, L)] = _key_to_f32(key0[sl])
        gidx[r, pl.ds(c, L)] = idx0[sl] + off
    pltpu.sync_copy(vals, conf_out.at[b])

    for c in range(NCH):
        rows = pl.ds(c * CH, CH)
        fcp = pltpu.async_copy(feat_hbm.at[gidx.at[c]], fbuf, fsem)
        acp = pltpu.async_copy(anch_hbm.at[gidx.at[c]], abuf, asem)
        fcp.wait()
        acp.wait()
        pltpu.sync_copy(fbuf, feat_out.at[b, rows])
        pltpu.sync_copy(abuf, anch_out.at[b, rows])
    rows = pl.ds(NCH * CH, REM)
    ridx = gidx.at[NCH, pl.ds(0, REM)]
    fcp = pltpu.async_copy(feat_hbm.at[ridx], fbuf.at[pl.ds(0, REM)], fsem)
    acp = pltpu.async_copy(anch_hbm.at[ridx], abuf.at[pl.ds(0, REM)], asem)
    fcp.wait()
    acp.wait()
    pltpu.sync_copy(fbuf.at[pl.ds(0, REM)], feat_out.at[b, rows])
    pltpu.sync_copy(abuf.at[pl.ds(0, REM)], anch_out.at[b, rows])


@jax.jit
def _run(conf3d, feat_flat, anch_flat, koff_arr):
    mesh = plsc.VectorSubcoreMesh(core_axis_name="c", subcore_axis_name="s")
    out_type = (
        jax.ShapeDtypeStruct((BS, 8, 128), jnp.float32),
        jax.ShapeDtypeStruct((BS, KP, D), jnp.float32),
        jax.ShapeDtypeStruct((BS, KP, 128), jnp.float32),
    )
    scratch = [
        pltpu.VMEM((N // 128, 128), jnp.float32),
        pltpu.VMEM((N,), jnp.int32),
        pltpu.VMEM((N,), jnp.int32),
        pltpu.VMEM((N,), jnp.int32),
        pltpu.VMEM((N,), jnp.int32),
        pltpu.VMEM((256,), jnp.int32),
        pltpu.VMEM((256,), jnp.int32),
        pltpu.VMEM((8, 128), jnp.float32),
        pltpu.VMEM((8, 128), jnp.int32),
        pltpu.VMEM((L,), jnp.int32),
        pltpu.VMEM((CH, D), jnp.float32),
        pltpu.VMEM((CH, 128), jnp.float32),
        pltpu.SemaphoreType.DMA,
        pltpu.SemaphoreType.DMA,
    ]
    f = pl.kernel(_sc_body, out_type=out_type, mesh=mesh,
                  scratch_types=scratch,
                  compiler_params=pltpu.CompilerParams(
                      needs_layout_passes=False))
    return f(conf3d, feat_flat, anch_flat, koff_arr)


def kernel(confidence, instance_feature, anchor, k):
    koff = jnp.asarray(k, jnp.int32) - np.int32(K)
    koff_arr = jnp.full((L,), koff, jnp.int32)
    conf3d = confidence.reshape(BS, N // 128, 128)
    feat_flat = instance_feature.reshape(BS * N, D)
    anch128 = jnp.pad(anchor.reshape(BS * N, AD), ((0, 0), (0, 128 - AD)))
    conf, feat, anch = _run(conf3d, feat_flat, anch128, koff_arr)
    return (conf.reshape(BS, 1024)[:, :K],
            feat[:, :K], anch[:, :K, :AD])
